# baseline (speedup 1.0000x reference)
import math

import jax
import jax.numpy as jnp
from jax import lax
from jax.experimental import pallas as pl
from jax.experimental.pallas import tpu as pltpu
from jax.experimental.pallas import tpu_sc as plsc

BASE_GSD = 10.0
HW_BLK = 256
B_BLK = 2


def _sc_gather_body(months, mtab, out, mons_v, mrows_v, si):
    sid = lax.axis_index("s")
    cid = lax.axis_index("c")
    nb = out.shape[0]

    @pl.when((sid < nb) & (cid == 0))
    def _():
        b = sid
        pltpu.sync_copy(months.at[b], mons_v)
        pltpu.async_copy(mtab.at[mons_v], mrows_v, si).wait()
        pltpu.sync_copy(mrows_v, out.at[b])


def _tc_dense_body(gsd_ref, a1n_ref, mrows_ref, x_ref, o_ref):
    f32 = jnp.float32
    gsd = gsd_ref[0, 0]
    hwb = pl.program_id(1)
    t, cg = 12, 4

    m48 = jnp.broadcast_to(
        mrows_ref[...][:, :t, None, :],
        (B_BLK, t, cg, 128)).reshape(B_BLK, t * cg, 128)
    a1 = a1n_ref[...][None] + m48

    hw = hwb * HW_BLK + lax.broadcasted_iota(jnp.int32, (HW_BLK, 8), 0)
    iv = (hw // 16).astype(f32) * gsd
    jv = (hw % 16).astype(f32) * gsd
    om8 = 1.0 / (10000.0 ** (
        lax.broadcasted_iota(jnp.int32, (HW_BLK, 8), 1).astype(f32) / 8.0))
    aj = jv * om8
    ai = iv * om8
    sp = jnp.concatenate([
        jnp.zeros((HW_BLK, 96), f32),
        jnp.sin(aj), jnp.cos(aj), jnp.sin(ai), jnp.cos(ai),
    ], axis=1)

    o_ref[...] = x_ref[...] + a1[:, None, :, :] + sp[None, :, None, :]


def kernel(per_modality_input_tokens, timestamps, channel_embed, patch_size,
           input_res):
    x = per_modality_input_tokens
    b, h, w, t, cg, D = x.shape
    f32 = jnp.float32
    xr = x.reshape(b, h * w, t * cg, D)

    om16 = 1.0 / (10000.0 ** (jnp.arange(16, dtype=f32) / 16.0))
    targ = jnp.arange(t, dtype=f32)[:, None] * om16[None, :]
    pos32 = jnp.concatenate([jnp.sin(targ), jnp.cos(targ)], axis=1)
    mang = jnp.arange(12, dtype=f32) / f32(12.0 / (2.0 * math.pi))
    mtab = jnp.concatenate([
        jnp.zeros((12, 64), f32),
        jnp.broadcast_to(jnp.sin(mang)[:, None], (12, 16)),
        jnp.broadcast_to(jnp.cos(mang)[:, None], (12, 16)),
        jnp.zeros((12, D - 96), f32),
    ], axis=1)

    ch48 = jnp.tile(channel_embed.astype(f32), (t, 1))
    pos48 = jnp.repeat(pos32, cg, axis=0)
    a1n = jnp.concatenate(
        [ch48, pos48, jnp.zeros((t * cg, 64), f32)], axis=1)

    months = jnp.zeros((b, 16), jnp.int32)
    months = months.at[:, :t].set(timestamps[:, 1, :].astype(jnp.int32))

    mesh = plsc.VectorSubcoreMesh(core_axis_name="c", subcore_axis_name="s")
    mrows = pl.kernel(
        _sc_gather_body, mesh=mesh,
        out_type=jax.ShapeDtypeStruct((b, 16, D), f32),
        scratch_types=[
            pltpu.VMEM((16,), jnp.int32),
            pltpu.VMEM((16, D), f32),
            pltpu.SemaphoreType.DMA,
        ],
    )(months, mtab)

    gsd = (jnp.asarray(input_res).astype(f32)
           * jnp.asarray(patch_size).astype(f32) / BASE_GSD).reshape(1, 1)
    out = pl.pallas_call(
        _tc_dense_body,
        grid=(b // B_BLK, (h * w) // HW_BLK),
        in_specs=[
            pl.BlockSpec(memory_space=pltpu.SMEM),
            pl.BlockSpec((t * cg, D), lambda bi, hi: (0, 0)),
            pl.BlockSpec((B_BLK, 16, D), lambda bi, hi: (bi, 0, 0)),
            pl.BlockSpec((B_BLK, HW_BLK, t * cg, D),
                         lambda bi, hi: (bi, hi, 0, 0)),
        ],
        out_specs=pl.BlockSpec((B_BLK, HW_BLK, t * cg, D),
                               lambda bi, hi: (bi, hi, 0, 0)),
        out_shape=jax.ShapeDtypeStruct(xr.shape, xr.dtype),
        compiler_params=pltpu.CompilerParams(
            dimension_semantics=("parallel", "parallel"),
            vmem_limit_bytes=100 * 1024 * 1024),
    )(gsd, a1n, mrows, xr)
    return out.reshape(b, h, w, t, cg, D)


# speedup vs baseline: 1.0079x; 1.0079x over previous
import math

import jax
import jax.numpy as jnp
from jax import lax
from jax.experimental import pallas as pl
from jax.experimental.pallas import tpu as pltpu
from jax.experimental.pallas import tpu_sc as plsc

BASE_GSD = 10.0
HW_BLK = 256
B_BLK = 1


def _sc_gather_body(months, mtab, out, mons_v, mrows_v, si):
    sid = lax.axis_index("s")
    cid = lax.axis_index("c")
    nb = out.shape[0]

    @pl.when((sid < nb) & (cid == 0))
    def _():
        b = sid
        pltpu.sync_copy(months.at[b], mons_v)
        pltpu.async_copy(mtab.at[mons_v], mrows_v, si).wait()
        pltpu.sync_copy(mrows_v, out.at[b])


def _tc_dense_body(gsd_ref, a1n_ref, mrows_ref, x_ref, o_ref):
    f32 = jnp.float32
    gsd = gsd_ref[0, 0]
    hwb = pl.program_id(1)
    t, cg = 12, 4

    m48 = jnp.broadcast_to(
        mrows_ref[0][:t, None, :], (t, cg, 128)).reshape(t * cg, 128)
    a1 = a1n_ref[...] + m48

    hw = hwb * HW_BLK + lax.broadcasted_iota(jnp.int32, (HW_BLK, 8), 0)
    iv = (hw // 16).astype(f32) * gsd
    jv = (hw % 16).astype(f32) * gsd
    om8 = 1.0 / (10000.0 ** (
        lax.broadcasted_iota(jnp.int32, (HW_BLK, 8), 1).astype(f32) / 8.0))
    aj = jv * om8
    ai = iv * om8
    sp = jnp.concatenate([
        jnp.zeros((HW_BLK, 96), f32),
        jnp.sin(aj), jnp.cos(aj), jnp.sin(ai), jnp.cos(ai),
    ], axis=1)

    o_ref[0] = x_ref[0] + a1[None, :, :] + sp[:, None, :]


def kernel(per_modality_input_tokens, timestamps, channel_embed, patch_size,
           input_res):
    x = per_modality_input_tokens
    b, h, w, t, cg, D = x.shape
    f32 = jnp.float32
    xr = x.reshape(b, h * w, t * cg, D)

    om16 = 1.0 / (10000.0 ** (jnp.arange(16, dtype=f32) / 16.0))
    targ = jnp.arange(t, dtype=f32)[:, None] * om16[None, :]
    pos32 = jnp.concatenate([jnp.sin(targ), jnp.cos(targ)], axis=1)
    mang = jnp.arange(12, dtype=f32) / f32(12.0 / (2.0 * math.pi))
    mtab = jnp.concatenate([
        jnp.zeros((12, 64), f32),
        jnp.broadcast_to(jnp.sin(mang)[:, None], (12, 16)),
        jnp.broadcast_to(jnp.cos(mang)[:, None], (12, 16)),
        jnp.zeros((12, D - 96), f32),
    ], axis=1)

    ch48 = jnp.tile(channel_embed.astype(f32), (t, 1))
    pos48 = jnp.repeat(pos32, cg, axis=0)
    a1n = jnp.concatenate(
        [ch48, pos48, jnp.zeros((t * cg, 64), f32)], axis=1)

    months = jnp.zeros((b, 16), jnp.int32)
    months = months.at[:, :t].set(timestamps[:, 1, :].astype(jnp.int32))

    mesh = plsc.VectorSubcoreMesh(core_axis_name="c", subcore_axis_name="s")
    mrows = pl.kernel(
        _sc_gather_body, mesh=mesh,
        out_type=jax.ShapeDtypeStruct((b, 16, D), f32),
        scratch_types=[
            pltpu.VMEM((16,), jnp.int32),
            pltpu.VMEM((16, D), f32),
            pltpu.SemaphoreType.DMA,
        ],
    )(months, mtab)

    gsd = (jnp.asarray(input_res).astype(f32)
           * jnp.asarray(patch_size).astype(f32) / BASE_GSD).reshape(1, 1)
    out = pl.pallas_call(
        _tc_dense_body,
        grid=(b // B_BLK, (h * w) // HW_BLK),
        in_specs=[
            pl.BlockSpec(memory_space=pltpu.SMEM),
            pl.BlockSpec((t * cg, D), lambda bi, hi: (0, 0)),
            pl.BlockSpec((1, 16, D), lambda bi, hi: (bi, 0, 0)),
            pl.BlockSpec((B_BLK, HW_BLK, t * cg, D),
                         lambda bi, hi: (bi, hi, 0, 0)),
        ],
        out_specs=pl.BlockSpec((B_BLK, HW_BLK, t * cg, D),
                               lambda bi, hi: (bi, hi, 0, 0)),
        out_shape=jax.ShapeDtypeStruct(xr.shape, xr.dtype),
        compiler_params=pltpu.CompilerParams(
            dimension_semantics=("parallel", "parallel")),
    )(gsd, a1n, mrows, xr)
    return out.reshape(b, h, w, t, cg, D)
